# Initial kernel scaffold; baseline (speedup 1.0000x reference)
import jax
import jax.numpy as jnp
from jax.experimental import pallas as pl


def kernel(x, edge_index, batch, node_type, prot_W, prot_b, prot_ln_g, prot_ln_b, lig_W, lig_b, lig_ln_g, lig_ln_b, node_type_emb, gcn_W0, gcn_b0, ln_g0, ln_b0, gcn_W1, gcn_b1, ln_g1, ln_b1, gcn_W2, gcn_b2, ln_g2, ln_b2, W1, b1, W2, b2):
    raise NotImplementedError("write your pallas kernel here")


# speedup vs baseline: 2.3467x; 2.3467x over previous
import functools

import jax
import jax.numpy as jnp
from jax import lax
from jax.experimental import pallas as pl
from jax.experimental.pallas import tpu as pltpu

N = 50000
E = 1600000
NODE_DIM = 1310
LIG_DIM = 36
HID = 256
ATT_DIM = 128
HEADS = 4
BATCH = 64

ROWS_BLK = 1000
N_BLKS = N // ROWS_BLK


def _ln(x, g, b, eps=1e-5):
    mu = jnp.mean(x, axis=-1, keepdims=True)
    var = jnp.var(x, axis=-1, keepdims=True)
    return (x - mu) * lax.rsqrt(var + eps) * g + b


def _input_proj_body(x_ref, nt_ref, prot_W_ref, prot_b_ref, prot_g_ref,
                     prot_bb_ref, lig_W_ref, lig_b_ref, lig_g_ref,
                     lig_bb_ref, emb_ref, out_ref):
    xa = x_ref[...]
    nt = nt_ref[0, 0, :].reshape(ROWS_BLK, 1)
    ph = jnp.dot(xa, prot_W_ref[...], preferred_element_type=jnp.float32)
    ph = jax.nn.relu(_ln(ph + prot_b_ref[...], prot_g_ref[...], prot_bb_ref[...]))
    lh = jnp.dot(xa[:, :LIG_DIM], lig_W_ref[...],
                 preferred_element_type=jnp.float32)
    lh = jax.nn.relu(_ln(lh + lig_b_ref[...], lig_g_ref[...], lig_bb_ref[...]))
    is_prot = nt == 0
    emb = jnp.where(is_prot, emb_ref[0:1, :], emb_ref[1:2, :])
    out_ref[...] = jnp.where(is_prot, ph, lh) + emb


def _input_proj(x, nt3, prot_W, prot_b, prot_g, prot_bb,
                lig_W, lig_b, lig_g, lig_bb, emb):
    full = lambda shape: pl.BlockSpec(shape, lambda i: (0,) * len(shape))
    return pl.pallas_call(
        _input_proj_body,
        grid=(N_BLKS,),
        in_specs=[
            pl.BlockSpec((ROWS_BLK, NODE_DIM), lambda i: (i, 0)),
            pl.BlockSpec((1, 1, ROWS_BLK), lambda i: (i, 0, 0)),
            full((NODE_DIM, HID)), full((HID,)), full((HID,)), full((HID,)),
            full((LIG_DIM, HID)), full((HID,)), full((HID,)), full((HID,)),
            full((2, HID)),
        ],
        out_specs=pl.BlockSpec((ROWS_BLK, HID), lambda i: (i, 0)),
        out_shape=jax.ShapeDtypeStruct((N, HID), jnp.float32),
    )(x, nt3, prot_W, prot_b, prot_g, prot_bb,
      lig_W, lig_b, lig_g, lig_bb, emb)


def kernel(x, edge_index, batch, node_type,
           prot_W, prot_b, prot_ln_g, prot_ln_b,
           lig_W, lig_b, lig_ln_g, lig_ln_b,
           node_type_emb,
           gcn_W0, gcn_b0, ln_g0, ln_b0,
           gcn_W1, gcn_b1, ln_g1, ln_b1,
           gcn_W2, gcn_b2, ln_g2, ln_b2,
           W1, b1, W2, b2):
    edge_index = edge_index.astype(jnp.int32)
    batch = batch.astype(jnp.int32)
    node_type = node_type.astype(jnp.int32)
    src = edge_index[0]
    dst = edge_index[1]

    nt3 = node_type.reshape(N_BLKS, 1, ROWS_BLK)
    h = _input_proj(x, nt3, prot_W, prot_b, prot_ln_g, prot_ln_b,
                    lig_W, lig_b, lig_ln_g, lig_ln_b, node_type_emb)

    deg = jnp.zeros((N,), jnp.float32).at[dst].add(1.0) + 1.0
    dinv = lax.rsqrt(deg)

    gcn = [(gcn_W0, gcn_b0, ln_g0, ln_b0),
           (gcn_W1, gcn_b1, ln_g1, ln_b1),
           (gcn_W2, gcn_b2, ln_g2, ln_b2)]
    for i, (W, b, g, bb) in enumerate(gcn):
        hw2 = (h @ W) * dinv[:, None]
        agg = jnp.zeros((N, HID), jnp.float32).at[dst].add(hw2[src])
        x_new = dinv[:, None] * (agg + hw2) + b
        if i > 0:
            x_new = x_new + h
        h = jax.nn.relu(_ln(x_new, g, bb))

    M = h
    protein = node_type == 0
    s = jnp.tanh(M @ W1 + b1) @ W2 + b2
    e = jnp.where(protein[:, None], jnp.exp(s), 0.0)
    denom = jax.ops.segment_sum(e, batch, num_segments=BATCH)
    denom = jnp.where(denom > 0, denom, 1.0)
    wbar = jnp.mean(e / denom[batch], axis=1)
    att_pool = jax.ops.segment_sum(M * wbar[:, None], batch,
                                   num_segments=BATCH)
    Mp = jnp.where(protein[:, None], M, 0.0)
    count = jax.ops.segment_sum(jnp.ones((N,), jnp.float32), batch,
                                num_segments=BATCH)
    summ = jax.ops.segment_sum(Mp, batch, num_segments=BATCH)
    global_pool = summ / jnp.maximum(count, 1.0)[:, None]
    return jnp.concatenate([att_pool, global_pool], axis=1)


# speedup vs baseline: 2.4474x; 1.0429x over previous
import functools

import jax
import jax.numpy as jnp
from jax import lax
from jax.experimental import pallas as pl
from jax.experimental.pallas import tpu as pltpu
from jax.experimental.pallas import tpu_sc as plsc

N = 50000
E = 1600000
NODE_DIM = 1310
LIG_DIM = 36
HID = 256
ATT_DIM = 128
HEADS = 4
BATCH = 64

ROWS_BLK = 1000
N_BLKS = N // ROWS_BLK

NC = 2
NS = 16
L = 16

E32 = E // (NC * NS)
BLK = 2000
NBLK32 = E32 // BLK
KB = 80
ZR = 200
NZCHUNK = N // ZR

_sc_mesh = plsc.VectorSubcoreMesh(core_axis_name="c", subcore_axis_name="s",
                                  num_cores=NC, num_subcores=NS)
_sc_params = pltpu.CompilerParams(needs_layout_passes=False)


def _ln(x, g, b, eps=1e-5):
    mu = jnp.mean(x, axis=-1, keepdims=True)
    var = jnp.var(x, axis=-1, keepdims=True)
    return (x - mu) * lax.rsqrt(var + eps) * g + b


def _input_proj_body(x_ref, nt_ref, prot_W_ref, prot_b_ref, prot_g_ref,
                     prot_bb_ref, lig_W_ref, lig_b_ref, lig_g_ref,
                     lig_bb_ref, emb_ref, out_ref):
    xa = x_ref[...]
    nt = nt_ref[0, 0, :].reshape(ROWS_BLK, 1)
    ph = jnp.dot(xa, prot_W_ref[...], preferred_element_type=jnp.float32)
    ph = jax.nn.relu(_ln(ph + prot_b_ref[...], prot_g_ref[...], prot_bb_ref[...]))
    lh = jnp.dot(xa[:, :LIG_DIM], lig_W_ref[...],
                 preferred_element_type=jnp.float32)
    lh = jax.nn.relu(_ln(lh + lig_b_ref[...], lig_g_ref[...], lig_bb_ref[...]))
    is_prot = nt == 0
    emb = jnp.where(is_prot, emb_ref[0:1, :], emb_ref[1:2, :])
    out_ref[...] = jnp.where(is_prot, ph, lh) + emb


def _input_proj(x, nt3, prot_W, prot_b, prot_g, prot_bb,
                lig_W, lig_b, lig_g, lig_bb, emb):
    full = lambda shape: pl.BlockSpec(shape, lambda i: (0,) * len(shape))
    return pl.pallas_call(
        _input_proj_body,
        grid=(N_BLKS,),
        in_specs=[
            pl.BlockSpec((ROWS_BLK, NODE_DIM), lambda i: (i, 0)),
            pl.BlockSpec((1, 1, ROWS_BLK), lambda i: (i, 0, 0)),
            full((NODE_DIM, HID)), full((HID,)), full((HID,)), full((HID,)),
            full((LIG_DIM, HID)), full((HID,)), full((HID,)), full((HID,)),
            full((2, HID)),
        ],
        out_specs=pl.BlockSpec((ROWS_BLK, HID), lambda i: (i, 0)),
        out_shape=jax.ShapeDtypeStruct((N, HID), jnp.float32),
    )(x, nt3, prot_W, prot_b, prot_g, prot_bb,
      lig_W, lig_b, lig_g, lig_bb, emb)


@functools.partial(
    pl.kernel,
    out_type=jax.ShapeDtypeStruct((NC * NS, N), jnp.float32),
    mesh=_sc_mesh,
    compiler_params=_sc_params,
    scratch_types=[
        pltpu.VMEM((N,), jnp.float32),
        pltpu.VMEM((BLK,), jnp.int32),
    ],
)
def _deg_kernel(dst_hbm, out_hbm, degbuf, dstblk):
    ci = lax.axis_index("c")
    si = lax.axis_index("s")
    tid = si * NC + ci
    ones = jnp.full((L,), 1.0, jnp.float32)

    def zero_body(i, _):
        degbuf[pl.ds(i * L, L)] = jnp.zeros((L,), jnp.float32)
        return 0
    lax.fori_loop(0, N // L, zero_body, 0)

    def blk_body(b, _):
        pltpu.sync_copy(dst_hbm.at[pl.ds(tid * E32 + b * BLK, BLK)], dstblk)

        def scan_body(i, _):
            d = dstblk[pl.ds(i * L, L)]
            plsc.addupdate_scatter(degbuf, [d], ones)
            return 0
        lax.fori_loop(0, BLK // L, scan_body, 0)
        return 0
    lax.fori_loop(0, NBLK32, blk_body, 0)

    pltpu.sync_copy(degbuf, out_hbm.at[tid])


def kernel(x, edge_index, batch, node_type,
           prot_W, prot_b, prot_ln_g, prot_ln_b,
           lig_W, lig_b, lig_ln_g, lig_ln_b,
           node_type_emb,
           gcn_W0, gcn_b0, ln_g0, ln_b0,
           gcn_W1, gcn_b1, ln_g1, ln_b1,
           gcn_W2, gcn_b2, ln_g2, ln_b2,
           W1, b1, W2, b2):
    edge_index = edge_index.astype(jnp.int32)
    batch = batch.astype(jnp.int32)
    node_type = node_type.astype(jnp.int32)
    src = edge_index[0]
    dst = edge_index[1]

    nt3 = node_type.reshape(N_BLKS, 1, ROWS_BLK)
    h = _input_proj(x, nt3, prot_W, prot_b, prot_ln_g, prot_ln_b,
                    lig_W, lig_b, lig_ln_g, lig_ln_b, node_type_emb)

    deg = _deg_kernel(dst).sum(axis=0) + 1.0
    dinv = lax.rsqrt(deg)

    gcn = [(gcn_W0, gcn_b0, ln_g0, ln_b0),
           (gcn_W1, gcn_b1, ln_g1, ln_b1),
           (gcn_W2, gcn_b2, ln_g2, ln_b2)]
    for i, (W, b, g, bb) in enumerate(gcn):
        hw2 = (h @ W) * dinv[:, None]
        agg = jnp.zeros((N, HID), jnp.float32).at[dst].add(hw2[src])
        x_new = dinv[:, None] * (agg + hw2) + b
        if i > 0:
            x_new = x_new + h
        h = jax.nn.relu(_ln(x_new, g, bb))

    M = h
    protein = node_type == 0
    s = jnp.tanh(M @ W1 + b1) @ W2 + b2
    e = jnp.where(protein[:, None], jnp.exp(s), 0.0)
    denom = jax.ops.segment_sum(e, batch, num_segments=BATCH)
    denom = jnp.where(denom > 0, denom, 1.0)
    wbar = jnp.mean(e / denom[batch], axis=1)
    att_pool = jax.ops.segment_sum(M * wbar[:, None], batch,
                                   num_segments=BATCH)
    Mp = jnp.where(protein[:, None], M, 0.0)
    count = jax.ops.segment_sum(jnp.ones((N,), jnp.float32), batch,
                                num_segments=BATCH)
    summ = jax.ops.segment_sum(Mp, batch, num_segments=BATCH)
    global_pool = summ / jnp.maximum(count, 1.0)[:, None]
    return jnp.concatenate([att_pool, global_pool], axis=1)
